# Initial kernel scaffold; baseline (speedup 1.0000x reference)
import jax
import jax.numpy as jnp
from jax.experimental import pallas as pl


def kernel(contexts, enc_ans, enc_ctx):
    raise NotImplementedError("write your pallas kernel here")


# speedup vs baseline: 1.5343x; 1.5343x over previous
import functools

import jax
import jax.numpy as jnp
import numpy as np
from jax import lax
from jax.experimental import pallas as pl
from jax.experimental.pallas import tpu as pltpu
from jax.experimental.pallas import tpu_sc as plsc

KTOP = 64
NQ = 64
V = 1000000
DIM = 32

BLK = 8192
NBLK = (V + BLK - 1) // BLK
SEGW = 128
NSEG = NBLK * SEGW
NSEGP = 16384
NCAND = KTOP * 64
NEG = float("-inf")
IBIG = 1 << 24


def _tc_body(ctx_ref, enc_ref, sims_ref, l1_ref):
    b = pl.program_id(0)
    ctx = ctx_ref[...].astype(jnp.bfloat16)
    blk = enc_ref[...].astype(jnp.bfloat16)
    s = jax.lax.dot_general(ctx, blk, (((1,), (1,)), ((), ())),
                            preferred_element_type=jnp.float32)
    sims_ref[...] = s
    l1_ref[...] = jnp.max(s.reshape(NQ, BLK // SEGW, SEGW), axis=1)

    @pl.when(b == NBLK - 1)
    def _():
        lim = V - (NBLK - 1) * BLK
        col = jax.lax.broadcasted_iota(jnp.int32, (NQ, BLK), 1)
        sm = jnp.where(col < lim, s, NEG)
        l1_ref[...] = jnp.max(sm.reshape(NQ, BLK // SEGW, SEGW), axis=1)


def _tc_pass(queries, enc):
    return pl.pallas_call(
        _tc_body,
        grid=(NBLK,),
        in_specs=[pl.BlockSpec((NQ, DIM), lambda b: (0, 0)),
                  pl.BlockSpec((BLK, DIM), lambda b: (b, 0))],
        out_specs=[pl.BlockSpec((NQ, BLK), lambda b: (0, b)),
                   pl.BlockSpec((NQ, SEGW), lambda b: (0, b))],
        out_shape=[jax.ShapeDtypeStruct((NQ, V), jnp.float32),
                   jax.ShapeDtypeStruct((NQ, NSEG), jnp.float32)],
    )(queries, enc)


def _i16():
    return lax.iota(jnp.int32, 16)


def _build(src, dst, ngroups):
    def g_body(g, c):
        def j_body(j, acc):
            return jnp.maximum(acc, src[pl.ds((g * 16 + j) * 16, 16)])
        acc = lax.fori_loop(0, 16, j_body,
                            jnp.full((16,), NEG, jnp.float32))
        dst[pl.ds(g * 16, 16)] = acc
        return c
    lax.fori_loop(0, ngroups, g_body, 0)


def _rebuild(src, dst, g):
    def j_body(j, acc):
        return jnp.maximum(acc, src[pl.ds((g * 16 + j) * 16, 16)])
    acc = lax.fori_loop(0, 16, j_body, jnp.full((16,), NEG, jnp.float32))
    dst[pl.ds(g * 16, 16)] = acc


def _find_elem(ref, nvregs, v):
    def body(g, best):
        vr = ref[pl.ds(g * 16, 16)]
        enc = jnp.where(vr == v, g * 16 + _i16(), IBIG)
        return jnp.minimum(best, jnp.min(enc))
    return lax.fori_loop(0, nvregs, body, np.int32(IBIG))


def _find_at_lane(ref, base, v, lane):
    def body(j, best):
        vr = ref[pl.ds((base + j) * 16, 16)]
        enc = jnp.where((vr == v) & (_i16() == lane), j, IBIG)
        return jnp.minimum(best, jnp.min(enc))
    return lax.fori_loop(0, 16, body, np.int32(IBIG))


def _kill(ref, vreg, lane):
    off = vreg * 16
    vr = ref[pl.ds(off, 16)]
    ref[pl.ds(off, 16)] = jnp.where(_i16() == lane, NEG, vr)


def _put_i32(ref, p, val):
    off = (p // 16) * 16
    vr = ref[pl.ds(off, 16)]
    ref[pl.ds(off, 16)] = jnp.where(_i16() == (p % 16), val, vr)


def _get_i32(ref, p):
    vr = ref[pl.ds((p // 16) * 16, 16)]
    return jnp.max(jnp.where(_i16() == (p % 16), vr, np.int32(-(1 << 30))))


def _pop3(l0, m1, m2):
    def top_body(h, acc):
        return jnp.maximum(acc, m2[pl.ds(h * 16, 16)])
    top = lax.fori_loop(0, 4, top_body, jnp.full((16,), NEG, jnp.float32))
    v = jnp.max(top)
    p2 = _find_elem(m2, 4, v)
    h, lane = p2 // 16, p2 % 16
    j = h * 16 + _find_at_lane(m1, h * 16, v, lane)
    i = j * 16 + _find_at_lane(l0, j * 16, v, lane)
    p = i * 16 + lane
    _kill(l0, i, lane)
    _rebuild(l0, m1, j)
    _rebuild(m1, m2, h)
    return v, p


def _pop2(c0, c1, c2):
    tv = c2[pl.ds(0, 16)]
    v = jnp.max(tv)
    lane = jnp.min(jnp.where(tv == v, _i16(), IBIG))
    j = _find_at_lane(c1, 0, v, lane)
    i = j * 16 + _find_at_lane(c0, j * 16, v, lane)
    p = i * 16 + lane
    _kill(c0, i, lane)
    _rebuild(c0, c1, j)
    _rebuild(c1, c2, 0)
    return v, p


def _select_row(r, l1_ref, sims_ref, l0, m1, m2, cidx, gidx, cval, c1, c2,
                winq, sem, eidx=None):
    pltpu.sync_copy(l1_ref.at[r], l0.at[pl.ds(0, NSEG)])

    def fill_body(i, c):
        l0[pl.ds(NSEG + i * 16, 16)] = jnp.full((16,), NEG, jnp.float32)
        return c
    lax.fori_loop(0, (NSEGP - NSEG) // 16, fill_body, 0)

    _build(l0, m1, 64)
    _build(m1, m2, 4)

    def a_body(t, c):
        _v, p = _pop3(l0, m1, m2)
        b = p // SEGW
        lcol = p % SEGW
        col0 = b * BLK + lcol

        def c_body(cc, c2_):
            col = col0 + (cc * 16 + _i16()) * SEGW
            cidx[pl.ds(t * 64 + cc * 16, 16)] = col
            gidx[pl.ds(t * 64 + cc * 16, 16)] = (
                r * V + jnp.minimum(col, V - 1))
            return c2_
        lax.fori_loop(0, 4, c_body, 0)
        return c
    lax.fori_loop(0, KTOP, a_body, 0)

    pltpu.async_copy(sims_ref.at[gidx], cval, sem).wait()

    def mask_body(i, c):
        cc = cidx[pl.ds(i * 16, 16)]
        vv = cval[pl.ds(i * 16, 16)]
        cval[pl.ds(i * 16, 16)] = jnp.where(cc < V, vv, NEG)
        return c
    lax.fori_loop(0, NCAND // 16, mask_body, 0)

    _build(cval, c1, 16)
    _build(c1, c2, 1)

    def b_body(t, c):
        _v, p = _pop2(cval, c1, c2)
        col = _get_i32(cidx, p)
        _put_i32(winq, t, col)
        if eidx is not None:
            eidx[pl.ds(t * 32, 16)] = col * 32 + _i16()
            eidx[pl.ds(t * 32 + 16, 16)] = col * 32 + 16 + _i16()
        return c
    lax.fori_loop(0, KTOP, b_body, 0)


def _sc_hop1_body(l1_ref, sims_ref, enc_ref, out_ref, l0, m1, m2, cidx,
                  gidx, cval, c1, c2, winq, eidx, rowsf, ncb, sem):
    wid = lax.axis_index("s") * 2 + lax.axis_index("c")

    def row_body(q, c):
        r = wid * 2 + q
        _select_row(r, l1_ref, sims_ref, l0, m1, m2, cidx, gidx, cval,
                    c1, c2, winq, sem, eidx=eidx)
        pltpu.async_copy(enc_ref.at[eidx], rowsf, sem).wait()

        def acc_body(k, accs):
            a0, a1 = accs
            return (a0 + rowsf[pl.ds(k * 32, 16)],
                    a1 + rowsf[pl.ds(k * 32 + 16, 16)])
        z = jnp.zeros((16,), jnp.float32)
        a0, a1 = lax.fori_loop(0, KTOP, acc_body, (z, z))
        ncb[pl.ds(0, 16)] = a0 * (1.0 / KTOP)
        ncb[pl.ds(16, 16)] = a1 * (1.0 / KTOP)
        pltpu.sync_copy(ncb, out_ref.at[r])
        return c
    lax.fori_loop(0, 2, row_body, 0)


def _sc_hop2_body(l1_ref, sims_ref, out_ref, l0, m1, m2, cidx, gidx, cval,
                  c1, c2, winq, sem):
    wid = lax.axis_index("s") * 2 + lax.axis_index("c")

    def row_body(q, c):
        r = wid * 2 + q
        _select_row(r, l1_ref, sims_ref, l0, m1, m2, cidx, gidx, cval,
                    c1, c2, winq, sem)
        pltpu.sync_copy(winq, out_ref.at[r])
        return c
    lax.fori_loop(0, 2, row_body, 0)


_MESH = plsc.VectorSubcoreMesh(core_axis_name="c", subcore_axis_name="s")

_COMMON_SCRATCH = [
    pltpu.VMEM((NSEGP,), jnp.float32),
    pltpu.VMEM((1024,), jnp.float32),
    pltpu.VMEM((64,), jnp.float32),
    pltpu.VMEM((NCAND,), jnp.int32),
    pltpu.VMEM((NCAND,), jnp.int32),
    pltpu.VMEM((NCAND,), jnp.float32),
    pltpu.VMEM((256,), jnp.float32),
    pltpu.VMEM((16,), jnp.float32),
    pltpu.VMEM((KTOP,), jnp.int32),
]

_SC_PARAMS = pltpu.CompilerParams(needs_layout_passes=False)

_sc_hop1 = pl.kernel(
    _sc_hop1_body, mesh=_MESH,
    compiler_params=_SC_PARAMS,
    out_type=jax.ShapeDtypeStruct((NQ, DIM), jnp.float32),
    scratch_types=_COMMON_SCRATCH + [
        pltpu.VMEM((KTOP * DIM,), jnp.int32),
        pltpu.VMEM((KTOP * DIM,), jnp.float32),
        pltpu.VMEM((DIM,), jnp.float32),
        pltpu.SemaphoreType.DMA,
    ],
)

_sc_hop2 = pl.kernel(
    _sc_hop2_body, mesh=_MESH,
    compiler_params=_SC_PARAMS,
    out_type=jax.ShapeDtypeStruct((NQ, KTOP), jnp.int32),
    scratch_types=_COMMON_SCRATCH + [pltpu.SemaphoreType.DMA],
)


def kernel(contexts, enc_ans, enc_ctx):
    sims2, l1a = _tc_pass(contexts, enc_ctx)
    new_contexts = _sc_hop1(l1a, sims2.reshape(-1), enc_ctx.reshape(-1))
    sims1, l1b = _tc_pass(new_contexts, enc_ans)
    return _sc_hop2(l1b, sims1.reshape(-1))


# speedup vs baseline: 14.6767x; 9.5655x over previous
import jax
import jax.numpy as jnp
import numpy as np
from jax import lax
from jax.experimental import pallas as pl
from jax.experimental.pallas import tpu as pltpu
from jax.experimental.pallas import tpu_sc as plsc

KTOP = 64
NQ = 64
V = 1000000
DIM = 32

BLK = 16384
NBLK = (V + BLK - 1) // BLK
SEG = 128
SPB = BLK // SEG
NSEG = NBLK * SPB
NSEGP = 8192
NCAND = KTOP * SEG
NEG = float("-inf")
IBIG = 1 << 24


def _tc_body(ctx_ref, enc_ref, sims_ref, l1_ref):
    b = pl.program_id(0)
    ctx = ctx_ref[...].astype(jnp.bfloat16)
    blk = enc_ref[...].astype(jnp.bfloat16)
    s = jax.lax.dot_general(ctx, blk, (((1,), (1,)), ((), ())),
                            preferred_element_type=jnp.float32)
    s3 = s.reshape(NQ, SPB, SEG)
    sims_ref[...] = s3
    l1_ref[...] = jnp.max(s3, axis=2)

    @pl.when(b == NBLK - 1)
    def _():
        lim = V - (NBLK - 1) * BLK
        col = jax.lax.broadcasted_iota(jnp.int32, (NQ, BLK), 1)
        sm = jnp.where(col < lim, s, NEG)
        l1_ref[...] = jnp.max(sm.reshape(NQ, SPB, SEG), axis=2)


def _tc_pass(queries, enc):
    return pl.pallas_call(
        _tc_body,
        grid=(NBLK,),
        in_specs=[pl.BlockSpec((NQ, DIM), lambda b: (0, 0)),
                  pl.BlockSpec((BLK, DIM), lambda b: (b, 0))],
        out_specs=[pl.BlockSpec((NQ, SPB, SEG), lambda b: (0, b, 0)),
                   pl.BlockSpec((NQ, SPB), lambda b: (0, b))],
        out_shape=[jax.ShapeDtypeStruct((NQ, NSEG, SEG), jnp.float32),
                   jax.ShapeDtypeStruct((NQ, NSEG), jnp.float32)],
    )(queries, enc)


def _i16():
    return lax.iota(jnp.int32, 16)


def _build(src, dst, ngroups):
    def g_body(g, c):
        def j_body(j, acc):
            return jnp.maximum(acc, src[pl.ds((g * 16 + j) * 16, 16)])
        acc = lax.fori_loop(0, 16, j_body,
                            jnp.full((16,), NEG, jnp.float32))
        dst[pl.ds(g * 16, 16)] = acc
        return c
    lax.fori_loop(0, ngroups, g_body, 0)


def _rebuild(src, dst, g):
    def j_body(j, acc):
        return jnp.maximum(acc, src[pl.ds((g * 16 + j) * 16, 16)])
    acc = lax.fori_loop(0, 16, j_body, jnp.full((16,), NEG, jnp.float32))
    dst[pl.ds(g * 16, 16)] = acc


def _find_elem(ref, nvregs, v):
    def body(g, best):
        vr = ref[pl.ds(g * 16, 16)]
        enc = jnp.where(vr == v, g * 16 + _i16(), IBIG)
        return jnp.minimum(best, jnp.min(enc))
    return lax.fori_loop(0, nvregs, body, np.int32(IBIG))


def _find_at_lane(ref, base, v, lane):
    def body(j, best):
        vr = ref[pl.ds((base + j) * 16, 16)]
        enc = jnp.where((vr == v) & (_i16() == lane), j, IBIG)
        return jnp.minimum(best, jnp.min(enc))
    return lax.fori_loop(0, 16, body, np.int32(IBIG))


def _kill(ref, vreg, lane):
    off = vreg * 16
    vr = ref[pl.ds(off, 16)]
    ref[pl.ds(off, 16)] = jnp.where(_i16() == lane, NEG, vr)


def _put_i32(ref, p, val):
    off = (p // 16) * 16
    vr = ref[pl.ds(off, 16)]
    ref[pl.ds(off, 16)] = jnp.where(_i16() == (p % 16), val, vr)


def _get_i32(ref, p):
    vr = ref[pl.ds((p // 16) * 16, 16)]
    return jnp.max(jnp.where(_i16() == (p % 16), vr, np.int32(-(1 << 30))))


def _pop(l0, m1, m2, n2):
    def top_body(h, acc):
        return jnp.maximum(acc, m2[pl.ds(h * 16, 16)])
    top = lax.fori_loop(0, n2, top_body, jnp.full((16,), NEG, jnp.float32))
    v = jnp.max(top)
    p2 = _find_elem(m2, n2, v)
    h, lane = p2 // 16, p2 % 16
    j = h * 16 + _find_at_lane(m1, h * 16, v, lane)
    i = j * 16 + _find_at_lane(l0, j * 16, v, lane)
    p = i * 16 + lane
    _kill(l0, i, lane)
    _rebuild(l0, m1, j)
    _rebuild(m1, m2, h)
    return v, p


def _select_row(r, l1_ref, sims_ref, l0, m1, m2, cidx, gidx, cval, c1, c2,
                winq, sem):
    roff = pl.multiple_of(r * NSEG, 8)
    pltpu.sync_copy(l1_ref.at[pl.ds(roff, NSEG)], l0.at[pl.ds(0, NSEG)])

    def fill_body(i, c):
        l0[pl.ds(NSEG + i * 16, 16)] = jnp.full((16,), NEG, jnp.float32)
        return c
    lax.fori_loop(0, (NSEGP - NSEG) // 16, fill_body, 0)

    _build(l0, m1, NSEGP // 256)
    _build(m1, m2, NSEGP // 4096)

    def a_body(t, c):
        _v, p = _pop(l0, m1, m2, NSEGP // 4096)
        base = p * SEG

        def c_body(cc, c2_):
            col = base + cc * 16 + _i16()
            gidx[pl.ds(t * SEG + cc * 16, 16)] = r * (NSEG * SEG) + col
            cidx[pl.ds(t * SEG + cc * 16, 16)] = jnp.where(col < V, col, -1)
            return c2_
        lax.fori_loop(0, SEG // 16, c_body, 0)
        return c
    lax.fori_loop(0, KTOP, a_body, 0)

    pltpu.async_copy(sims_ref.at[gidx], cval, sem).wait()

    def mask_body(i, c):
        cc = cidx[pl.ds(i * 16, 16)]
        vv = cval[pl.ds(i * 16, 16)]
        cval[pl.ds(i * 16, 16)] = jnp.where(cc >= 0, vv, NEG)
        return c
    lax.fori_loop(0, NCAND // 16, mask_body, 0)

    _build(cval, c1, NCAND // 256)
    _build(c1, c2, NCAND // 4096)

    def b_body(t, c):
        _v, p = _pop(cval, c1, c2, NCAND // 4096)
        _put_i32(winq, t, _get_i32(cidx, p))
        return c
    lax.fori_loop(0, KTOP, b_body, 0)


def _sc_hop1_body(l1_ref, sims_ref, enc_ref, out_ref, l0, m1, m2, cidx,
                  gidx, cval, c1, c2, winq, bande, ncb, sem):
    wid = lax.axis_index("s") * 2 + lax.axis_index("c")

    def row_body(q, c):
        r = wid * 2 + q
        _select_row(r, l1_ref, sims_ref, l0, m1, m2, cidx, gidx, cval,
                    c1, c2, winq, sem)

        def fetch_body(t, c2_):
            col = _get_i32(winq, t)
            colb = pl.multiple_of(8 * (col // 8), 8)
            pltpu.async_copy(enc_ref.at[pl.ds(colb, 8)], bande.at[t], sem)
            return c2_
        lax.fori_loop(0, KTOP, fetch_body, 0)

        def drain_body(t, c2_):
            pltpu.make_async_copy(enc_ref.at[pl.ds(0, 8)],
                                  bande.at[t], sem).wait()
            return c2_
        lax.fori_loop(0, KTOP, drain_body, 0)

        def acc_body(k, accs):
            a0, a1 = accs
            rm = _get_i32(winq, k) % 8
            return (a0 + bande[k, rm, pl.ds(0, 16)],
                    a1 + bande[k, rm, pl.ds(16, 16)])
        z = jnp.zeros((16,), jnp.float32)
        a0, a1 = lax.fori_loop(0, KTOP, acc_body, (z, z))
        ncb[pl.ds(0, 16)] = a0 * (1.0 / KTOP)
        ncb[pl.ds(16, 16)] = a1 * (1.0 / KTOP)
        pltpu.sync_copy(ncb, out_ref.at[pl.ds(pl.multiple_of(r * DIM, 8),
                                              DIM)])
        return c
    lax.fori_loop(0, 2, row_body, 0)


def _sc_hop2_body(l1_ref, sims_ref, out_ref, l0, m1, m2, cidx, gidx,
                  cval, c1, c2, winq, sem):
    wid = lax.axis_index("s") * 2 + lax.axis_index("c")

    def row_body(q, c):
        r = wid * 2 + q
        _select_row(r, l1_ref, sims_ref, l0, m1, m2, cidx, gidx, cval,
                    c1, c2, winq, sem)
        pltpu.sync_copy(winq, out_ref.at[pl.ds(pl.multiple_of(r * KTOP, 8),
                                               KTOP)])
        return c
    lax.fori_loop(0, 2, row_body, 0)


_MESH = plsc.VectorSubcoreMesh(core_axis_name="c", subcore_axis_name="s")

_SC_PARAMS = pltpu.CompilerParams(needs_layout_passes=False)

_COMMON_SCRATCH = [
    pltpu.VMEM((NSEGP,), jnp.float32),
    pltpu.VMEM((NSEGP // 16,), jnp.float32),
    pltpu.VMEM((NSEGP // 256,), jnp.float32),
    pltpu.VMEM((NCAND,), jnp.int32),
    pltpu.VMEM((NCAND,), jnp.int32),
    pltpu.VMEM((NCAND,), jnp.float32),
    pltpu.VMEM((NCAND // 16,), jnp.float32),
    pltpu.VMEM((NCAND // 256,), jnp.float32),
    pltpu.VMEM((KTOP,), jnp.int32),
]

_sc_hop1 = pl.kernel(
    _sc_hop1_body, mesh=_MESH,
    compiler_params=_SC_PARAMS,
    out_type=jax.ShapeDtypeStruct((NQ * DIM,), jnp.float32),
    scratch_types=_COMMON_SCRATCH + [
        pltpu.VMEM((KTOP, 8, DIM), jnp.float32),
        pltpu.VMEM((DIM,), jnp.float32),
        pltpu.SemaphoreType.DMA,
    ],
)

_sc_hop2 = pl.kernel(
    _sc_hop2_body, mesh=_MESH,
    compiler_params=_SC_PARAMS,
    out_type=jax.ShapeDtypeStruct((NQ * KTOP,), jnp.int32),
    scratch_types=_COMMON_SCRATCH + [pltpu.SemaphoreType.DMA],
)


def kernel(contexts, enc_ans, enc_ctx):
    sims2, l1a = _tc_pass(contexts, enc_ctx)
    nc_flat = _sc_hop1(l1a.reshape(-1), sims2.reshape(-1), enc_ctx)
    new_contexts = nc_flat.reshape(NQ, DIM)
    sims1, l1b = _tc_pass(new_contexts, enc_ans)
    ixs_flat = _sc_hop2(l1b.reshape(-1), sims1.reshape(-1))
    return ixs_flat.reshape(NQ, KTOP)


# speedup vs baseline: 14.9516x; 1.0187x over previous
import jax
import jax.numpy as jnp
import numpy as np
from jax import lax
from jax.experimental import pallas as pl
from jax.experimental.pallas import tpu as pltpu
from jax.experimental.pallas import tpu_sc as plsc

KTOP = 64
NQ = 64
V = 1000000
DIM = 32

BLK = 16384
NBLK = (V + BLK - 1) // BLK
SEG = 128
SPB = BLK // SEG
NSEG = NBLK * SPB
NSEGP = 8192
NCAND = KTOP * SEG
NEG = float("-inf")
IBIG = 1 << 24


def _tc_body(ctx_ref, enc_ref, sims_ref, l1_ref):
    b = pl.program_id(0)
    ctx = ctx_ref[...].astype(jnp.bfloat16)
    blk = enc_ref[...].astype(jnp.bfloat16)
    s = jax.lax.dot_general(ctx, blk, (((1,), (1,)), ((), ())),
                            preferred_element_type=jnp.float32)
    s3 = s.reshape(NQ, SPB, SEG)
    sims_ref[...] = s3
    l1_ref[...] = jnp.max(s3, axis=2)

    @pl.when(b == NBLK - 1)
    def _():
        lim = V - (NBLK - 1) * BLK
        col = jax.lax.broadcasted_iota(jnp.int32, (NQ, BLK), 1)
        sm = jnp.where(col < lim, s, NEG)
        l1_ref[...] = jnp.max(sm.reshape(NQ, SPB, SEG), axis=2)


def _tc_pass(queries, enc):
    return pl.pallas_call(
        _tc_body,
        grid=(NBLK,),
        in_specs=[pl.BlockSpec((NQ, DIM), lambda b: (0, 0)),
                  pl.BlockSpec((BLK, DIM), lambda b: (b, 0))],
        out_specs=[pl.BlockSpec((NQ, SPB, SEG), lambda b: (0, b, 0)),
                   pl.BlockSpec((NQ, SPB), lambda b: (0, b))],
        out_shape=[jax.ShapeDtypeStruct((NQ, NSEG, SEG), jnp.float32),
                   jax.ShapeDtypeStruct((NQ, NSEG), jnp.float32)],
    )(queries, enc)


def _i16():
    return lax.iota(jnp.int32, 16)


def _build(src, dst, ngroups):
    def g_body(g, c):
        def j_body(j, acc):
            return jnp.maximum(acc, src[pl.ds((g * 16 + j) * 16, 16)])
        acc = lax.fori_loop(0, 16, j_body,
                            jnp.full((16,), NEG, jnp.float32))
        dst[pl.ds(g * 16, 16)] = acc
        return c
    lax.fori_loop(0, ngroups, g_body, 0)


def _rebuild(src, dst, g):
    def j_body(j, acc):
        return jnp.maximum(acc, src[pl.ds((g * 16 + j) * 16, 16)])
    acc = lax.fori_loop(0, 16, j_body, jnp.full((16,), NEG, jnp.float32))
    dst[pl.ds(g * 16, 16)] = acc


def _find_elem(ref, nvregs, v):
    def body(g, best):
        vr = ref[pl.ds(g * 16, 16)]
        enc = jnp.where(vr == v, g * 16 + _i16(), IBIG)
        return jnp.minimum(best, jnp.min(enc))
    return lax.fori_loop(0, nvregs, body, np.int32(IBIG))


def _find_at_lane(ref, base, v, lane):
    def body(j, best):
        vr = ref[pl.ds((base + j) * 16, 16)]
        enc = jnp.where((vr == v) & (_i16() == lane), j, IBIG)
        return jnp.minimum(best, jnp.min(enc))
    return lax.fori_loop(0, 16, body, np.int32(IBIG))


def _kill(ref, vreg, lane):
    off = vreg * 16
    vr = ref[pl.ds(off, 16)]
    ref[pl.ds(off, 16)] = jnp.where(_i16() == lane, NEG, vr)


def _put_i32(ref, p, val):
    off = (p // 16) * 16
    vr = ref[pl.ds(off, 16)]
    ref[pl.ds(off, 16)] = jnp.where(_i16() == (p % 16), val, vr)


def _get_i32(ref, p):
    vr = ref[pl.ds((p // 16) * 16, 16)]
    return jnp.max(jnp.where(_i16() == (p % 16), vr, np.int32(-(1 << 30))))


def _pop(l0, m1, m2, n2):
    def top_body(h, acc):
        return jnp.maximum(acc, m2[pl.ds(h * 16, 16)])
    top = lax.fori_loop(0, n2, top_body, jnp.full((16,), NEG, jnp.float32))
    v = jnp.max(top)
    p2 = _find_elem(m2, n2, v)
    h, lane = p2 // 16, p2 % 16
    j = h * 16 + _find_at_lane(m1, h * 16, v, lane)
    i = j * 16 + _find_at_lane(l0, j * 16, v, lane)
    p = i * 16 + lane
    _kill(l0, i, lane)
    _rebuild(l0, m1, j)
    _rebuild(m1, m2, h)
    return v, p


def _ld2(ref, x):
    return ref[x // 8, pl.ds((x % 8) * 16, 16)]


def _build2(src2, dst, ngroups):
    def g_body(g, c):
        def j_body(j, acc):
            return jnp.maximum(acc, _ld2(src2, g * 16 + j))
        acc = lax.fori_loop(0, 16, j_body,
                            jnp.full((16,), NEG, jnp.float32))
        dst[pl.ds(g * 16, 16)] = acc
        return c
    lax.fori_loop(0, ngroups, g_body, 0)


def _rebuild2(src2, dst, g):
    def j_body(j, acc):
        return jnp.maximum(acc, _ld2(src2, g * 16 + j))
    acc = lax.fori_loop(0, 16, j_body, jnp.full((16,), NEG, jnp.float32))
    dst[pl.ds(g * 16, 16)] = acc


def _find_at_lane2(ref2, base, v, lane):
    def body(j, best):
        vr = _ld2(ref2, base + j)
        enc = jnp.where((vr == v) & (_i16() == lane), j, IBIG)
        return jnp.minimum(best, jnp.min(enc))
    return lax.fori_loop(0, 16, body, np.int32(IBIG))


def _pop_b(c0, c1, c2, n2):
    def top_body(h, acc):
        return jnp.maximum(acc, c2[pl.ds(h * 16, 16)])
    top = lax.fori_loop(0, n2, top_body, jnp.full((16,), NEG, jnp.float32))
    v = jnp.max(top)
    p2 = _find_elem(c2, n2, v)
    h, lane = p2 // 16, p2 % 16
    j = h * 16 + _find_at_lane(c1, h * 16, v, lane)
    i = j * 16 + _find_at_lane2(c0, j * 16, v, lane)
    p = i * 16 + lane
    vr = _ld2(c0, i)
    c0[i // 8, pl.ds((i % 8) * 16, 16)] = jnp.where(_i16() == lane, NEG, vr)
    _rebuild2(c0, c1, j)
    _rebuild(c1, c2, h)
    return v, p


def _select_row(r, l1_ref, sims_ref, l0, m1, m2, cidx, sidx, cval2, c1, c2,
                winq, sem):
    roff = pl.multiple_of(r * NSEG, 8)
    pltpu.sync_copy(l1_ref.at[pl.ds(roff, NSEG)], l0.at[pl.ds(0, NSEG)])

    def fill_body(i, c):
        l0[pl.ds(NSEG + i * 16, 16)] = jnp.full((16,), NEG, jnp.float32)
        return c
    lax.fori_loop(0, (NSEGP - NSEG) // 16, fill_body, 0)

    _build(l0, m1, NSEGP // 256)
    _build(m1, m2, NSEGP // 4096)

    def a_body(t, c):
        _v, p = _pop(l0, m1, m2, NSEGP // 4096)
        base = p * SEG
        _put_i32(sidx, t, r * NSEG + p)

        def c_body(cc, c2_):
            col = base + cc * 16 + _i16()
            cidx[pl.ds(t * SEG + cc * 16, 16)] = jnp.where(col < V, col, -1)
            return c2_
        lax.fori_loop(0, SEG // 16, c_body, 0)
        return c
    lax.fori_loop(0, KTOP, a_body, 0)

    pltpu.async_copy(sims_ref.at[sidx], cval2, sem).wait()

    def mask_body(i, c):
        cc = cidx[pl.ds(i * 16, 16)]
        vv = _ld2(cval2, i)
        cval2[i // 8, pl.ds((i % 8) * 16, 16)] = jnp.where(cc >= 0, vv, NEG)
        return c
    lax.fori_loop(0, NCAND // 16, mask_body, 0)

    _build2(cval2, c1, NCAND // 256)
    _build(c1, c2, NCAND // 4096)

    def b_body(t, c):
        _v, p = _pop_b(cval2, c1, c2, NCAND // 4096)
        _put_i32(winq, t, _get_i32(cidx, p))
        return c
    lax.fori_loop(0, KTOP, b_body, 0)


def _sc_hop1_body(l1_ref, sims_ref, enc_ref, out_ref, l0, m1, m2, cidx,
                  sidx, cval, c1, c2, winq, bande, ncb, sem):
    wid = lax.axis_index("s") * 2 + lax.axis_index("c")

    def row_body(q, c):
        r = wid * 2 + q
        _select_row(r, l1_ref, sims_ref, l0, m1, m2, cidx, sidx, cval,
                    c1, c2, winq, sem)

        def fetch_body(t, c2_):
            col = _get_i32(winq, t)
            colb = pl.multiple_of(8 * (col // 8), 8)
            pltpu.async_copy(enc_ref.at[pl.ds(colb, 8)], bande.at[t], sem)
            return c2_
        lax.fori_loop(0, KTOP, fetch_body, 0)

        def drain_body(t, c2_):
            pltpu.make_async_copy(enc_ref.at[pl.ds(0, 8)],
                                  bande.at[t], sem).wait()
            return c2_
        lax.fori_loop(0, KTOP, drain_body, 0)

        def acc_body(k, accs):
            a0, a1 = accs
            rm = _get_i32(winq, k) % 8
            return (a0 + bande[k, rm, pl.ds(0, 16)],
                    a1 + bande[k, rm, pl.ds(16, 16)])
        z = jnp.zeros((16,), jnp.float32)
        a0, a1 = lax.fori_loop(0, KTOP, acc_body, (z, z))
        ncb[pl.ds(0, 16)] = a0 * (1.0 / KTOP)
        ncb[pl.ds(16, 16)] = a1 * (1.0 / KTOP)
        pltpu.sync_copy(ncb, out_ref.at[pl.ds(pl.multiple_of(r * DIM, 8),
                                              DIM)])
        return c
    lax.fori_loop(0, 2, row_body, 0)


def _sc_hop2_body(l1_ref, sims_ref, out_ref, l0, m1, m2, cidx, sidx,
                  cval, c1, c2, winq, sem):
    wid = lax.axis_index("s") * 2 + lax.axis_index("c")

    def row_body(q, c):
        r = wid * 2 + q
        _select_row(r, l1_ref, sims_ref, l0, m1, m2, cidx, sidx, cval,
                    c1, c2, winq, sem)
        pltpu.sync_copy(winq, out_ref.at[pl.ds(pl.multiple_of(r * KTOP, 8),
                                               KTOP)])
        return c
    lax.fori_loop(0, 2, row_body, 0)


_MESH = plsc.VectorSubcoreMesh(core_axis_name="c", subcore_axis_name="s")

_SC_PARAMS = pltpu.CompilerParams(needs_layout_passes=False)

_COMMON_SCRATCH = [
    pltpu.VMEM((NSEGP,), jnp.float32),
    pltpu.VMEM((NSEGP // 16,), jnp.float32),
    pltpu.VMEM((NSEGP // 256,), jnp.float32),
    pltpu.VMEM((NCAND,), jnp.int32),
    pltpu.VMEM((KTOP,), jnp.int32),
    pltpu.VMEM((KTOP, SEG), jnp.float32),
    pltpu.VMEM((NCAND // 16,), jnp.float32),
    pltpu.VMEM((NCAND // 256,), jnp.float32),
    pltpu.VMEM((KTOP,), jnp.int32),
]

_sc_hop1 = pl.kernel(
    _sc_hop1_body, mesh=_MESH,
    compiler_params=_SC_PARAMS,
    out_type=jax.ShapeDtypeStruct((NQ * DIM,), jnp.float32),
    scratch_types=_COMMON_SCRATCH + [
        pltpu.VMEM((KTOP, 8, DIM), jnp.float32),
        pltpu.VMEM((DIM,), jnp.float32),
        pltpu.SemaphoreType.DMA,
    ],
)

_sc_hop2 = pl.kernel(
    _sc_hop2_body, mesh=_MESH,
    compiler_params=_SC_PARAMS,
    out_type=jax.ShapeDtypeStruct((NQ * KTOP,), jnp.int32),
    scratch_types=_COMMON_SCRATCH + [pltpu.SemaphoreType.DMA],
)


def kernel(contexts, enc_ans, enc_ctx):
    sims2, l1a = _tc_pass(contexts, enc_ctx)
    nc_flat = _sc_hop1(l1a.reshape(-1), sims2.reshape(NQ * NSEG, SEG),
                       enc_ctx)
    new_contexts = nc_flat.reshape(NQ, DIM)
    sims1, l1b = _tc_pass(new_contexts, enc_ans)
    ixs_flat = _sc_hop2(l1b.reshape(-1), sims1.reshape(NQ * NSEG, SEG))
    return ixs_flat.reshape(NQ, KTOP)


# speedup vs baseline: 14.9598x; 1.0006x over previous
import jax
import jax.numpy as jnp
import numpy as np
from jax import lax
from jax.experimental import pallas as pl
from jax.experimental.pallas import tpu as pltpu
from jax.experimental.pallas import tpu_sc as plsc

KTOP = 64
NQ = 64
V = 1000000
DIM = 32

BLK = 16384
NBLK = (V + BLK - 1) // BLK
SEG = 128
SPB = BLK // SEG
NSEG = NBLK * SPB
NSEGP = 8192
NCAND = KTOP * SEG
NEG = float("-inf")
IBIG = 1 << 24


def _tc_body(ctx_ref, enc_ref, sims_ref, l1_ref):
    b = pl.program_id(0)
    ctx = ctx_ref[...].astype(jnp.bfloat16)
    blk = enc_ref[...].astype(jnp.bfloat16)
    s = jax.lax.dot_general(ctx, blk, (((1,), (1,)), ((), ())),
                            preferred_element_type=jnp.float32)
    s3 = s.reshape(NQ, SPB, SEG)
    sims_ref[...] = s3
    l1_ref[...] = jnp.max(s3, axis=2)

    @pl.when(b == NBLK - 1)
    def _():
        lim = V - (NBLK - 1) * BLK
        col = jax.lax.broadcasted_iota(jnp.int32, (NQ, BLK), 1)
        sm = jnp.where(col < lim, s, NEG)
        l1_ref[...] = jnp.max(sm.reshape(NQ, SPB, SEG), axis=2)


def _tc_pass(queries, enc):
    return pl.pallas_call(
        _tc_body,
        grid=(NBLK,),
        in_specs=[pl.BlockSpec((NQ, DIM), lambda b: (0, 0)),
                  pl.BlockSpec((BLK, DIM), lambda b: (b, 0))],
        out_specs=[pl.BlockSpec((NQ, SPB, SEG), lambda b: (0, b, 0)),
                   pl.BlockSpec((NQ, SPB), lambda b: (0, b))],
        out_shape=[jax.ShapeDtypeStruct((NQ, NSEG, SEG), jnp.float32),
                   jax.ShapeDtypeStruct((NQ, NSEG), jnp.float32)],
    )(queries, enc)


def _i16():
    return lax.iota(jnp.int32, 16)


def _build(src, dst, ngroups):
    def g_body(g, c):
        def j_body(j, acc):
            return jnp.maximum(acc, src[pl.ds((g * 16 + j) * 16, 16)])
        acc = lax.fori_loop(0, 16, j_body,
                            jnp.full((16,), NEG, jnp.float32))
        dst[pl.ds(g * 16, 16)] = acc
        return c
    lax.fori_loop(0, ngroups, g_body, 0)


def _rebuild(src, dst, g):
    def j_body(j, acc):
        return jnp.maximum(acc, src[pl.ds((g * 16 + j) * 16, 16)])
    acc = lax.fori_loop(0, 16, j_body, jnp.full((16,), NEG, jnp.float32))
    dst[pl.ds(g * 16, 16)] = acc


def _find_elem(ref, nvregs, v):
    def body(g, best):
        vr = ref[pl.ds(g * 16, 16)]
        enc = jnp.where(vr == v, g * 16 + _i16(), IBIG)
        return jnp.minimum(best, jnp.min(enc))
    return lax.fori_loop(0, nvregs, body, np.int32(IBIG))


def _find_at_lane(ref, base, v, lane):
    def body(j, best):
        vr = ref[pl.ds((base + j) * 16, 16)]
        enc = jnp.where((vr == v) & (_i16() == lane), j, IBIG)
        return jnp.minimum(best, jnp.min(enc))
    return lax.fori_loop(0, 16, body, np.int32(IBIG))


def _kill(ref, vreg, lane):
    off = vreg * 16
    vr = ref[pl.ds(off, 16)]
    ref[pl.ds(off, 16)] = jnp.where(_i16() == lane, NEG, vr)


def _put_i32(ref, p, val):
    off = (p // 16) * 16
    vr = ref[pl.ds(off, 16)]
    ref[pl.ds(off, 16)] = jnp.where(_i16() == (p % 16), val, vr)


def _get_i32(ref, p):
    vr = ref[pl.ds((p // 16) * 16, 16)]
    return jnp.max(jnp.where(_i16() == (p % 16), vr, np.int32(-(1 << 30))))


def _pop(l0, m1, m2, n2):
    def top_body(h, acc):
        return jnp.maximum(acc, m2[pl.ds(h * 16, 16)])
    top = lax.fori_loop(0, n2, top_body, jnp.full((16,), NEG, jnp.float32))
    v = jnp.max(top)
    p2 = _find_elem(m2, n2, v)
    h, lane = p2 // 16, p2 % 16
    j = h * 16 + _find_at_lane(m1, h * 16, v, lane)
    i = j * 16 + _find_at_lane(l0, j * 16, v, lane)
    p = i * 16 + lane
    _kill(l0, i, lane)
    _rebuild(l0, m1, j)
    _rebuild(m1, m2, h)
    return v, p


def _ld2(ref, x):
    return ref[x // 8, pl.ds((x % 8) * 16, 16)]


def _build2(src2, dst, ngroups):
    def g_body(g, c):
        def j_body(j, acc):
            return jnp.maximum(acc, _ld2(src2, g * 16 + j))
        acc = lax.fori_loop(0, 16, j_body,
                            jnp.full((16,), NEG, jnp.float32))
        dst[pl.ds(g * 16, 16)] = acc
        return c
    lax.fori_loop(0, ngroups, g_body, 0)


def _rebuild2(src2, dst, g):
    def j_body(j, acc):
        return jnp.maximum(acc, _ld2(src2, g * 16 + j))
    acc = lax.fori_loop(0, 16, j_body, jnp.full((16,), NEG, jnp.float32))
    dst[pl.ds(g * 16, 16)] = acc


def _find_at_lane2(ref2, base, v, lane):
    def body(j, best):
        vr = _ld2(ref2, base + j)
        enc = jnp.where((vr == v) & (_i16() == lane), j, IBIG)
        return jnp.minimum(best, jnp.min(enc))
    return lax.fori_loop(0, 16, body, np.int32(IBIG))


def _pop_b(c0, c1, c2, n2):
    def top_body(h, acc):
        return jnp.maximum(acc, c2[pl.ds(h * 16, 16)])
    top = lax.fori_loop(0, n2, top_body, jnp.full((16,), NEG, jnp.float32))
    v = jnp.max(top)
    p2 = _find_elem(c2, n2, v)
    h, lane = p2 // 16, p2 % 16
    j = h * 16 + _find_at_lane(c1, h * 16, v, lane)
    i = j * 16 + _find_at_lane2(c0, j * 16, v, lane)
    p = i * 16 + lane
    vr = _ld2(c0, i)
    c0[i // 8, pl.ds((i % 8) * 16, 16)] = jnp.where(_i16() == lane, NEG, vr)
    _rebuild2(c0, c1, j)
    _rebuild(c1, c2, h)
    return v, p


def _select_row(r, l1_ref, sims_ref, l0, m1, m2, cidx, sidx, cval2, c1, c2,
                winq, sem):
    roff = pl.multiple_of(r * NSEG, 8)
    pltpu.sync_copy(l1_ref.at[pl.ds(roff, NSEG)], l0.at[pl.ds(0, NSEG)])

    def fill_body(i, c):
        l0[pl.ds(NSEG + i * 16, 16)] = jnp.full((16,), NEG, jnp.float32)
        return c
    lax.fori_loop(0, (NSEGP - NSEG) // 16, fill_body, 0)

    _build(l0, m1, NSEGP // 256)
    _build(m1, m2, NSEGP // 4096)

    def a_body(t, c):
        _v, p = _pop(l0, m1, m2, NSEGP // 4096)
        base = p * SEG
        _put_i32(sidx, t, r * NSEG + p)

        def c_body(cc, c2_):
            col = base + cc * 16 + _i16()
            cidx[pl.ds(t * SEG + cc * 16, 16)] = jnp.where(col < V, col, -1)
            return c2_
        lax.fori_loop(0, SEG // 16, c_body, 0)
        return c
    lax.fori_loop(0, KTOP, a_body, 0)

    pltpu.async_copy(sims_ref.at[sidx], cval2, sem).wait()

    def mask_body(i, c):
        cc = cidx[pl.ds(i * 16, 16)]
        vv = _ld2(cval2, i)
        cval2[i // 8, pl.ds((i % 8) * 16, 16)] = jnp.where(cc >= 0, vv, NEG)
        return c
    lax.fori_loop(0, NCAND // 16, mask_body, 0)

    _build2(cval2, c1, NCAND // 256)
    _build(c1, c2, NCAND // 4096)

    def b_body(t, c):
        _v, p = _pop_b(cval2, c1, c2, NCAND // 4096)
        _put_i32(winq, t, _get_i32(cidx, p))
        return c
    lax.fori_loop(0, KTOP, b_body, 0)


def _sc_hop1_body(l1_ref, sims_ref, enc_ref, out_ref, l0, m1, m2, cidx,
                  sidx, cval, c1, c2, winq, bande, ncb, sem):
    wid = lax.axis_index("s") * 2 + lax.axis_index("c")

    def row_body(q, c):
        r = wid * 2 + q
        _select_row(r, l1_ref, sims_ref, l0, m1, m2, cidx, sidx, cval,
                    c1, c2, winq, sem)

        def fetch_body(t, c2_):
            col = _get_i32(winq, t)
            colb = pl.multiple_of(8 * (col // 8), 8)
            pltpu.async_copy(enc_ref.at[pl.ds(colb, 8)], bande.at[t], sem)
            return c2_
        lax.fori_loop(0, KTOP, fetch_body, 0)

        def drain_body(t, c2_):
            pltpu.make_async_copy(enc_ref.at[pl.ds(0, 8)],
                                  bande.at[t], sem).wait()
            return c2_
        lax.fori_loop(0, KTOP, drain_body, 0)

        def acc_body(k, accs):
            a0, a1 = accs
            rm = _get_i32(winq, k) % 8
            return (a0 + bande[k, rm, pl.ds(0, 16)],
                    a1 + bande[k, rm, pl.ds(16, 16)])
        z = jnp.zeros((16,), jnp.float32)
        a0, a1 = lax.fori_loop(0, KTOP, acc_body, (z, z))
        ncb[pl.ds(0, 16)] = a0 * (1.0 / KTOP)
        ncb[pl.ds(16, 16)] = a1 * (1.0 / KTOP)
        pltpu.sync_copy(ncb, out_ref.at[pl.ds(pl.multiple_of(r * DIM, 8),
                                              DIM)])
        return c
    lax.fori_loop(0, 2, row_body, 0)


def _sc_hop2_body(l1_ref, sims_ref, out_ref, l0, m1, m2, cidx, sidx,
                  cval, c1, c2, winq, sem):
    wid = lax.axis_index("s") * 2 + lax.axis_index("c")

    def row_body(q, c):
        r = wid * 2 + q
        _select_row(r, l1_ref, sims_ref, l0, m1, m2, cidx, sidx, cval,
                    c1, c2, winq, sem)
        pltpu.sync_copy(winq, out_ref.at[pl.ds(pl.multiple_of(r * KTOP, 8),
                                               KTOP)])
        return c
    lax.fori_loop(0, 2, row_body, 0)


_MESH = plsc.VectorSubcoreMesh(core_axis_name="c", subcore_axis_name="s")

_SC_PARAMS = pltpu.CompilerParams(needs_layout_passes=False,
                                 use_tc_tiling_on_sc=True)

_COMMON_SCRATCH = [
    pltpu.VMEM((NSEGP,), jnp.float32),
    pltpu.VMEM((NSEGP // 16,), jnp.float32),
    pltpu.VMEM((NSEGP // 256,), jnp.float32),
    pltpu.VMEM((NCAND,), jnp.int32),
    pltpu.VMEM((KTOP,), jnp.int32),
    pltpu.VMEM((KTOP, SEG), jnp.float32),
    pltpu.VMEM((NCAND // 16,), jnp.float32),
    pltpu.VMEM((NCAND // 256,), jnp.float32),
    pltpu.VMEM((KTOP,), jnp.int32),
]

_sc_hop1 = pl.kernel(
    _sc_hop1_body, mesh=_MESH,
    compiler_params=_SC_PARAMS,
    out_type=jax.ShapeDtypeStruct((NQ * DIM,), jnp.float32),
    scratch_types=_COMMON_SCRATCH + [
        pltpu.VMEM((KTOP, 8, DIM), jnp.float32),
        pltpu.VMEM((DIM,), jnp.float32),
        pltpu.SemaphoreType.DMA,
    ],
)

_sc_hop2 = pl.kernel(
    _sc_hop2_body, mesh=_MESH,
    compiler_params=_SC_PARAMS,
    out_type=jax.ShapeDtypeStruct((NQ * KTOP,), jnp.int32),
    scratch_types=_COMMON_SCRATCH + [pltpu.SemaphoreType.DMA],
)


def kernel(contexts, enc_ans, enc_ctx):
    sims2, l1a = _tc_pass(contexts, enc_ctx)
    nc_flat = _sc_hop1(l1a.reshape(-1), sims2.reshape(NQ * NSEG, SEG),
                       enc_ctx)
    new_contexts = nc_flat.reshape(NQ, DIM)
    sims1, l1b = _tc_pass(new_contexts, enc_ans)
    ixs_flat = _sc_hop2(l1b.reshape(-1), sims1.reshape(NQ * NSEG, SEG))
    return ixs_flat.reshape(NQ, KTOP)
